# Initial kernel scaffold; baseline (speedup 1.0000x reference)
import jax
import jax.numpy as jnp
from jax.experimental import pallas as pl


def kernel(x, batch, weight, bias):
    raise NotImplementedError("write your pallas kernel here")


# speedup vs baseline: 10.6363x; 10.6363x over previous
import jax
import jax.numpy as jnp
from jax import lax
from jax.experimental import pallas as pl
from jax.experimental.pallas import tpu as pltpu

_EPS = 1e-05
_SEG = 64
_TD = 960
_BN = 1000


def _onehot(bvec, rows):
    seg_iota = lax.broadcasted_iota(jnp.int32, (rows, _SEG), 1)
    return (bvec == seg_iota).astype(jnp.float32)


def _p1_body(x_ref, b_ref, colsum_ref, aux_ref):
    @pl.when(pl.program_id(0) == 0)
    def _():
        colsum_ref[...] = jnp.zeros_like(colsum_ref)
        aux_ref[...] = jnp.zeros_like(aux_ref)

    x = x_ref[...]
    oh = _onehot(b_ref[0], x.shape[0])
    dn = (((0,), (0,)), ((), ()))
    colsum_ref[...] += lax.dot_general(
        oh, x, dn, preferred_element_type=jnp.float32,
        precision=lax.Precision.HIGHEST)
    x0 = x[:, :256]
    rowsq = jnp.sum(x0 * x0, axis=1, keepdims=True)
    aux = jnp.concatenate(
        [jnp.ones_like(rowsq), rowsq,
         jnp.zeros((x.shape[0], 126), jnp.float32)], axis=1)
    aux_ref[...] += lax.dot_general(
        oh, aux, dn, preferred_element_type=jnp.float32,
        precision=lax.Precision.HIGHEST)


def _p2_body(x_ref, b_ref, colsum_ref, aux_ref, w_ref, bb_ref, o_ref,
             a_scr, b_scr):
    @pl.when(pl.program_id(0) == 0)
    def _():
        S = colsum_ref[...]
        cnt = jnp.maximum(aux_ref[:, 0:1], 1.0)
        sumsq0 = aux_ref[:, 1:2]
        ci = lax.broadcasted_iota(jnp.int32, (_SEG, _TD), 1)
        is0 = ci < 256
        is1 = (ci >= 256) & (ci < 640)
        is2 = ci >= 640
        sum0 = jnp.sum(jnp.where(is0, S, 0.0), axis=1, keepdims=True)
        mean0 = sum0 / (cnt * 256.0)
        norm0 = jnp.maximum(sumsq0 / (cnt * 256.0) - mean0 * mean0, 0.0)
        inv0 = 1.0 / (jnp.sqrt(norm0) + _EPS)
        mean = jnp.where(is0, mean0, 0.0)
        d1 = (ci - 256) % 3
        for d in range(3):
            m = is1 & (d1 == d)
            md = jnp.sum(jnp.where(m, S, 0.0), axis=1, keepdims=True) / (cnt * 128.0)
            mean = mean + jnp.where(m, md, 0.0)
        d2 = (ci - 640) % 5
        for d in range(5):
            m = is2 & (d2 == d)
            md = jnp.sum(jnp.where(m, S, 0.0), axis=1, keepdims=True) / (cnt * 64.0)
            mean = mean + jnp.where(m, md, 0.0)
        scale = jnp.where(is0, inv0, 1.0) * w_ref[...]
        a_scr[...] = scale
        b_scr[...] = bb_ref[...] - mean * scale

    x = x_ref[...]
    oh = _onehot(b_ref[0], x.shape[0])
    arows = jnp.dot(oh, a_scr[...], preferred_element_type=jnp.float32,
                    precision=lax.Precision.HIGHEST)
    brows = jnp.dot(oh, b_scr[...], preferred_element_type=jnp.float32,
                    precision=lax.Precision.HIGHEST)
    o_ref[...] = x * arows + brows


def kernel(x, batch, weight, bias):
    n, td = x.shape
    assert td == _TD and n % _BN == 0
    nb = n // _BN
    batch3 = batch.astype(jnp.int32).reshape(nb, _BN, 1)
    wcol = jnp.concatenate(
        [weight[:256], jnp.repeat(weight[256:384], 3),
         jnp.repeat(weight[384:448], 5)]).reshape(1, _TD)
    bcol = jnp.concatenate(
        [bias[:256], jnp.zeros((704,), bias.dtype)]).reshape(1, _TD)

    colsum, aux = pl.pallas_call(
        _p1_body,
        grid=(nb,),
        in_specs=[
            pl.BlockSpec((_BN, _TD), lambda i: (i, 0)),
            pl.BlockSpec((1, _BN, 1), lambda i: (i, 0, 0)),
        ],
        out_specs=[
            pl.BlockSpec((_SEG, _TD), lambda i: (0, 0)),
            pl.BlockSpec((_SEG, 128), lambda i: (0, 0)),
        ],
        out_shape=[
            jax.ShapeDtypeStruct((_SEG, _TD), jnp.float32),
            jax.ShapeDtypeStruct((_SEG, 128), jnp.float32),
        ],
        compiler_params=pltpu.CompilerParams(
            dimension_semantics=("arbitrary",)),
    )(x, batch3)

    out = pl.pallas_call(
        _p2_body,
        grid=(nb,),
        in_specs=[
            pl.BlockSpec((_BN, _TD), lambda i: (i, 0)),
            pl.BlockSpec((1, _BN, 1), lambda i: (i, 0, 0)),
            pl.BlockSpec((_SEG, _TD), lambda i: (0, 0)),
            pl.BlockSpec((_SEG, 128), lambda i: (0, 0)),
            pl.BlockSpec((1, _TD), lambda i: (0, 0)),
            pl.BlockSpec((1, _TD), lambda i: (0, 0)),
        ],
        out_specs=pl.BlockSpec((_BN, _TD), lambda i: (i, 0)),
        out_shape=jax.ShapeDtypeStruct((n, _TD), jnp.float32),
        scratch_shapes=[
            pltpu.VMEM((_SEG, _TD), jnp.float32),
            pltpu.VMEM((_SEG, _TD), jnp.float32),
        ],
        compiler_params=pltpu.CompilerParams(
            dimension_semantics=("arbitrary",)),
    )(x, batch3, colsum, aux, wcol, bcol)
    return out


# speedup vs baseline: 15.5642x; 1.4633x over previous
import jax
import jax.numpy as jnp
from jax import lax
from jax.experimental import pallas as pl
from jax.experimental.pallas import tpu as pltpu

_EPS = 1e-05
_SEG = 64
_TD = 960
_BN = 2000


def _onehot(bvec, rows):
    seg_iota = lax.broadcasted_iota(jnp.int32, (rows, _SEG), 1)
    return (bvec == seg_iota).astype(jnp.float32)


def _p1_body(x_ref, b_ref, colsum_ref, aux_ref):
    @pl.when(pl.program_id(0) == 0)
    def _():
        colsum_ref[...] = jnp.zeros_like(colsum_ref)
        aux_ref[...] = jnp.zeros_like(aux_ref)

    x = x_ref[...]
    oh = _onehot(b_ref[0], x.shape[0])
    dn = (((0,), (0,)), ((), ()))
    colsum_ref[...] += lax.dot_general(
        oh, x, dn, preferred_element_type=jnp.float32)
    x0 = x[:, :256]
    rowsq = jnp.sum(x0 * x0, axis=1, keepdims=True)
    aux = jnp.concatenate(
        [jnp.ones_like(rowsq), rowsq,
         jnp.zeros((x.shape[0], 126), jnp.float32)], axis=1)
    aux_ref[...] += lax.dot_general(
        oh, aux, dn, preferred_element_type=jnp.float32)


def _p2_body(x_ref, b_ref, colsum_ref, aux_ref, w_ref, bb_ref, o_ref,
             mean_scr, iv_scr):
    @pl.when(pl.program_id(0) == 0)
    def _():
        S = colsum_ref[...]
        cnt = jnp.maximum(aux_ref[:, 0:1], 1.0)
        sumsq0 = aux_ref[:, 1:2]
        ci = lax.broadcasted_iota(jnp.int32, (_SEG, _TD), 1)
        is0 = ci < 256
        is1 = (ci >= 256) & (ci < 640)
        is2 = ci >= 640
        sum0 = jnp.sum(jnp.where(is0, S, 0.0), axis=1, keepdims=True)
        mean0 = sum0 / (cnt * 256.0)
        norm0 = jnp.maximum(sumsq0 / (cnt * 256.0) - mean0 * mean0, 0.0)
        inv0 = 1.0 / (jnp.sqrt(norm0) + _EPS)
        mean = jnp.where(is0, mean0, 0.0)
        d1 = (ci - 256) % 3
        for d in range(3):
            m = is1 & (d1 == d)
            md = jnp.sum(jnp.where(m, S, 0.0), axis=1, keepdims=True) / (cnt * 128.0)
            mean = mean + jnp.where(m, md, 0.0)
        d2 = (ci - 640) % 5
        for d in range(5):
            m = is2 & (d2 == d)
            md = jnp.sum(jnp.where(m, S, 0.0), axis=1, keepdims=True) / (cnt * 64.0)
            mean = mean + jnp.where(m, md, 0.0)
        mean_scr[...] = mean
        inv_hi = inv0.astype(jnp.bfloat16).astype(jnp.float32)
        inv_lo = inv0 - inv_hi
        li = lax.broadcasted_iota(jnp.int32, (_SEG, 128), 1)
        iv_scr[...] = (jnp.where(li == 0, inv_hi, 0.0)
                       + jnp.where(li == 1, inv_lo, 0.0))

    x = x_ref[...]
    oh = _onehot(b_ref[0], x.shape[0])
    meanr = jnp.dot(oh, mean_scr[...], preferred_element_type=jnp.float32)
    ivr = jnp.dot(oh, iv_scr[...], preferred_element_type=jnp.float32)
    invr = ivr[:, 0:1] + ivr[:, 1:2]
    xm = x - meanr
    w = w_ref[...]
    out0 = xm[:, :256] * (invr * w[:, :256]) + bb_ref[:, :256]
    out12 = xm[:, 256:] * w[:, 256:]
    o_ref[:, :256] = out0
    o_ref[:, 256:] = out12


def kernel(x, batch, weight, bias):
    n, td = x.shape
    assert td == _TD and n % _BN == 0
    nb = n // _BN
    batch3 = batch.astype(jnp.int32).reshape(nb, _BN, 1)
    wcol = jnp.concatenate(
        [weight[:256], jnp.repeat(weight[256:384], 3),
         jnp.repeat(weight[384:448], 5)]).reshape(1, _TD)
    bcol = jnp.concatenate(
        [bias[:256], jnp.zeros((704,), bias.dtype)]).reshape(1, _TD)

    colsum, aux = pl.pallas_call(
        _p1_body,
        grid=(nb,),
        in_specs=[
            pl.BlockSpec((_BN, _TD), lambda i: (i, 0)),
            pl.BlockSpec((1, _BN, 1), lambda i: (i, 0, 0)),
        ],
        out_specs=[
            pl.BlockSpec((_SEG, _TD), lambda i: (0, 0)),
            pl.BlockSpec((_SEG, 128), lambda i: (0, 0)),
        ],
        out_shape=[
            jax.ShapeDtypeStruct((_SEG, _TD), jnp.float32),
            jax.ShapeDtypeStruct((_SEG, 128), jnp.float32),
        ],
        compiler_params=pltpu.CompilerParams(
            dimension_semantics=("arbitrary",)),
    )(x, batch3)

    out = pl.pallas_call(
        _p2_body,
        grid=(nb,),
        in_specs=[
            pl.BlockSpec((_BN, _TD), lambda i: (i, 0)),
            pl.BlockSpec((1, _BN, 1), lambda i: (i, 0, 0)),
            pl.BlockSpec((_SEG, _TD), lambda i: (0, 0)),
            pl.BlockSpec((_SEG, 128), lambda i: (0, 0)),
            pl.BlockSpec((1, _TD), lambda i: (0, 0)),
            pl.BlockSpec((1, _TD), lambda i: (0, 0)),
        ],
        out_specs=pl.BlockSpec((_BN, _TD), lambda i: (i, 0)),
        out_shape=jax.ShapeDtypeStruct((n, _TD), jnp.float32),
        scratch_shapes=[
            pltpu.VMEM((_SEG, _TD), jnp.float32),
            pltpu.VMEM((_SEG, 128), jnp.float32),
        ],
        compiler_params=pltpu.CompilerParams(
            dimension_semantics=("arbitrary",)),
    )(x, batch3, colsum, aux, wcol, bcol)
    return out
